# baseline (speedup 1.0000x reference)
import functools

import jax
import jax.numpy as jnp
from jax import lax
from jax.experimental import pallas as pl
from jax.experimental.pallas import tpu as pltpu
from jax.experimental.pallas import tpu_sc as plsc

_EPS_BN = 1e-5
_EPS_D = 1e-8


def _p0_body(nsteps, N, uf_ref, Wu_ref, bu_ref, gu_ref, betau_ref,
             A_ref, c_ref, s1_ref, s2_ref):
    i = pl.program_id(0) * pl.num_programs(1) + pl.program_id(1)
    blk = uf_ref[0]

    @pl.when(i == 0)
    def _init():
        s1_ref[...] = jnp.zeros_like(s1_ref)
        s2_ref[...] = jnp.zeros_like(s2_ref)

    s1_ref[...] += jnp.sum(blk, axis=1, keepdims=True)
    s2_ref[...] += lax.dot_general(blk, blk, (((1,), (1,)), ((), ())),
                                   preferred_element_type=jnp.float32)

    @pl.when(i == nsteps - 1)
    def _fin():
        Wu = Wu_ref[...]
        mean_uf = s1_ref[...] / N
        mu_lin = jnp.dot(Wu, mean_uf, preferred_element_type=jnp.float32)
        e2 = jnp.sum(
            jnp.dot(Wu, s2_ref[...], preferred_element_type=jnp.float32) * Wu,
            axis=1, keepdims=True) / N
        var = e2 - mu_lin * mu_lin
        scale = gu_ref[...] * lax.rsqrt(var + _EPS_BN)
        A_ref[...] = Wu * scale
        c_ref[...] = betau_ref[...] - mu_lin * scale


def _p1_body(nsteps, N, m,
             un_ref, kn_ref, uf_ref, kf_ref,
             A_ref, cu_ref, W1_ref, b1_ref, g1_ref, beta1_ref,
             y1_ref, sc1_ref, sh1_ref, s1_ref, s2_ref):
    step = pl.program_id(0) * pl.num_programs(1) + pl.program_id(1)
    U = un_ref[0]
    K = kn_ref[0]
    e0 = K[:, 0:1] - U[0:1, :]
    e1 = K[:, 1:2] - U[1:2, :]
    e2 = K[:, 2:3] - U[2:3, :]
    d2 = e0 * e0 + e1 * e1 + e2 * e2

    rio = lax.broadcasted_iota(jnp.int32, d2.shape, 0)
    kk = (lax.bitcast_convert_type(d2, jnp.int32) & jnp.int32(-1024)) | rio
    vals, idxs = [], []
    for _ in range(3):
        kmin = jnp.min(kk, axis=0, keepdims=True)
        idxs.append(kmin & jnp.int32(1023))
        vals.append(lax.bitcast_convert_type(kmin & jnp.int32(-1024),
                                             jnp.float32))
        kk = jnp.where(kk == kmin, jnp.int32(2147483647), kk)

    r = [1.0 / (v + _EPS_D) for v in vals]
    norm = r[0] + r[1] + r[2]
    w = [ri / norm for ri in r]

    onehot = (jnp.where(rio == idxs[0], w[0], 0.0)
              + jnp.where(rio == idxs[1], w[1], 0.0)
              + jnp.where(rio == idxs[2], w[2], 0.0))
    interp = jnp.dot(kf_ref[0], onehot,
                     preferred_element_type=jnp.float32)

    x = interp + jnp.dot(A_ref[...], uf_ref[0],
                         preferred_element_type=jnp.float32) + cu_ref[...]
    y1 = jnp.dot(W1_ref[...], x,
                 preferred_element_type=jnp.float32) + b1_ref[...]
    y1_ref[0] = y1

    @pl.when(step == 0)
    def _init():
        s1_ref[...] = jnp.zeros_like(s1_ref)
        s2_ref[...] = jnp.zeros_like(s2_ref)

    s1_ref[...] += jnp.sum(y1, axis=1, keepdims=True)
    s2_ref[...] += jnp.sum(y1 * y1, axis=1, keepdims=True)

    @pl.when(step == nsteps - 1)
    def _fin():
        mean = s1_ref[...] / N
        var = s2_ref[...] / N - mean * mean
        scale = g1_ref[...] * lax.rsqrt(var + _EPS_BN)
        sc1_ref[...] = scale
        sh1_ref[...] = beta1_ref[...] - mean * scale


def _p1a_body(m, un_ref, kn_ref, gidx_ref, w_ref):
    b = pl.program_id(0)
    U = un_ref[0]
    K = kn_ref[0]
    e0 = K[:, 0:1] - U[0:1, :]
    e1 = K[:, 1:2] - U[1:2, :]
    e2 = K[:, 2:3] - U[2:3, :]
    d2 = e0 * e0 + e1 * e1 + e2 * e2

    rio = lax.broadcasted_iota(jnp.int32, d2.shape, 0)
    kk = (lax.bitcast_convert_type(d2, jnp.int32) & jnp.int32(-1024)) | rio
    vals, idxs = [], []
    for _ in range(3):
        kmin = jnp.min(kk, axis=0, keepdims=True)
        idxs.append(kmin & jnp.int32(1023))
        vals.append(lax.bitcast_convert_type(kmin & jnp.int32(-1024),
                                             jnp.float32))
        kk = jnp.where(kk == kmin, jnp.int32(2147483647), kk)

    r = [1.0 / (v + _EPS_D) for v in vals]
    norm = r[0] + r[1] + r[2]
    gidx_ref[...] = jnp.concatenate(idxs, axis=0) + b * m
    w_ref[...] = jnp.concatenate([ri / norm for ri in r], axis=0)


_SC_NW = 32
_SC_G = 64


def _make_sc_interp(N, C2):
    QW = N // _SC_NW
    G = _SC_G
    mesh = plsc.VectorSubcoreMesh(core_axis_name="c", subcore_axis_name="s")
    f32, i32 = jnp.float32, jnp.int32

    @functools.partial(
        pl.kernel, mesh=mesh,
        out_type=jax.ShapeDtypeStruct((N, C2), f32),
        scratch_types=[
            pltpu.VMEM((G,), i32), pltpu.VMEM((G,), i32), pltpu.VMEM((G,), i32),
            pltpu.VMEM((G + 16,), f32), pltpu.VMEM((G + 16,), f32),
            pltpu.VMEM((G + 16,), f32),
            pltpu.VMEM((G, C2), f32), pltpu.VMEM((G, C2), f32),
            pltpu.VMEM((G, C2), f32), pltpu.VMEM((G, C2), f32),
            pltpu.SemaphoreType.DMA,
        ],
    )
    def sc_interp(tab_hbm, gidx_hbm, w_hbm, out_hbm,
                  i0_v, i1_v, i2_v, w0_v, w1_v, w2_v,
                  r0_v, r1_v, r2_v, acc_v, sem):
        wid = lax.axis_index("s") * 2 + lax.axis_index("c")
        base = wid * QW

        def chunk(ci, carry):
            q0 = base + ci * G
            pltpu.sync_copy(gidx_hbm.at[0, pl.ds(q0, G)], i0_v)
            pltpu.sync_copy(gidx_hbm.at[1, pl.ds(q0, G)], i1_v)
            pltpu.sync_copy(gidx_hbm.at[2, pl.ds(q0, G)], i2_v)
            pltpu.sync_copy(w_hbm.at[0, pl.ds(q0, G)], w0_v.at[pl.ds(0, G)])
            pltpu.sync_copy(w_hbm.at[1, pl.ds(q0, G)], w1_v.at[pl.ds(0, G)])
            pltpu.sync_copy(w_hbm.at[2, pl.ds(q0, G)], w2_v.at[pl.ds(0, G)])
            pltpu.async_copy(tab_hbm.at[i0_v], r0_v, sem).wait()
            pltpu.async_copy(tab_hbm.at[i1_v], r1_v, sem).wait()
            pltpu.async_copy(tab_hbm.at[i2_v], r2_v, sem).wait()

            def per_q(q, c2):
                w0 = w0_v[pl.ds(q, 16)][0]
                w1 = w1_v[pl.ds(q, 16)][0]
                w2 = w2_v[pl.ds(q, 16)][0]
                for c in range(C2 // 16):
                    sl = pl.ds(c * 16, 16)
                    acc_v[q, sl] = (w0 * r0_v[q, sl] + w1 * r1_v[q, sl]
                                    + w2 * r2_v[q, sl])
                return c2

            lax.fori_loop(0, G, per_q, 0)
            pltpu.sync_copy(acc_v, out_hbm.at[pl.ds(q0, G)])
            return carry

        lax.fori_loop(0, QW // G, chunk, 0)

    return sc_interp


def _p1b_body(nsteps, N, it_ref, uf_ref,
              A_ref, cu_ref, W1_ref, b1_ref, g1_ref, beta1_ref,
              y1_ref, sc1_ref, sh1_ref, s1_ref, s2_ref):
    step = pl.program_id(0) * pl.num_programs(1) + pl.program_id(1)
    interp = jnp.transpose(it_ref[...])
    x = interp + jnp.dot(A_ref[...], uf_ref[0],
                         preferred_element_type=jnp.float32) + cu_ref[...]
    y1 = jnp.dot(W1_ref[...], x,
                 preferred_element_type=jnp.float32) + b1_ref[...]
    y1_ref[0] = y1

    @pl.when(step == 0)
    def _init():
        s1_ref[...] = jnp.zeros_like(s1_ref)
        s2_ref[...] = jnp.zeros_like(s2_ref)

    s1_ref[...] += jnp.sum(y1, axis=1, keepdims=True)
    s2_ref[...] += jnp.sum(y1 * y1, axis=1, keepdims=True)

    @pl.when(step == nsteps - 1)
    def _fin():
        mean = s1_ref[...] / N
        var = s2_ref[...] / N - mean * mean
        scale = g1_ref[...] * lax.rsqrt(var + _EPS_BN)
        sc1_ref[...] = scale
        sh1_ref[...] = beta1_ref[...] - mean * scale


def _p2_body(nsteps, N, y1_ref, sc1_ref, sh1_ref, W2_ref, b2_ref,
             g2_ref, beta2_ref, y2_ref, sc2_ref, sh2_ref, s1_ref, s2_ref):
    step = pl.program_id(0) * pl.num_programs(1) + pl.program_id(1)
    x1 = jnp.maximum(y1_ref[0] * sc1_ref[...] + sh1_ref[...], 0.0)
    y2 = jnp.dot(W2_ref[...], x1,
                 preferred_element_type=jnp.float32) + b2_ref[...]
    y2_ref[0] = y2

    @pl.when(step == 0)
    def _init():
        s1_ref[...] = jnp.zeros_like(s1_ref)
        s2_ref[...] = jnp.zeros_like(s2_ref)

    s1_ref[...] += jnp.sum(y2, axis=1, keepdims=True)
    s2_ref[...] += jnp.sum(y2 * y2, axis=1, keepdims=True)

    @pl.when(step == nsteps - 1)
    def _fin():
        mean = s1_ref[...] / N
        var = s2_ref[...] / N - mean * mean
        scale = g2_ref[...] * lax.rsqrt(var + _EPS_BN)
        sc2_ref[...] = scale
        sh2_ref[...] = beta2_ref[...] - mean * scale


def _p3_body(y2_ref, sc2_ref, sh2_ref, out_ref):
    out_ref[0] = jnp.maximum(y2_ref[0] * sc2_ref[...] + sh2_ref[...], 0.0)


def kernel(unknown, known, unknow_feats, known_feats, Wu, bu, gu, betau,
           W1, b1, g1, beta1, W2, b2, g2, beta2):
    B, n, _ = unknown.shape
    m = known.shape[1]
    C1 = unknow_feats.shape[1]
    C2 = known_feats.shape[1]
    N = B * n
    f32 = jnp.float32

    un_t = jnp.transpose(unknown, (0, 2, 1))
    col = lambda v: v.reshape(-1, 1)

    NB0 = 4096
    g0 = (B, n // NB0)
    A_u, c_u = pl.pallas_call(
        functools.partial(_p0_body, g0[0] * g0[1], float(N)),
        grid=g0,
        in_specs=[
            pl.BlockSpec((1, C1, NB0), lambda b, j: (b, 0, j)),
            pl.BlockSpec((C2, C1), lambda b, j: (0, 0)),
            pl.BlockSpec((C2, 1), lambda b, j: (0, 0)),
            pl.BlockSpec((C2, 1), lambda b, j: (0, 0)),
            pl.BlockSpec((C2, 1), lambda b, j: (0, 0)),
        ],
        out_specs=[
            pl.BlockSpec((C2, C1), lambda b, j: (0, 0)),
            pl.BlockSpec((C2, 1), lambda b, j: (0, 0)),
        ],
        out_shape=[
            jax.ShapeDtypeStruct((C2, C1), f32),
            jax.ShapeDtypeStruct((C2, 1), f32),
        ],
        scratch_shapes=[
            pltpu.VMEM((C1, 1), f32),
            pltpu.VMEM((C1, C1), f32),
        ],
    )(unknow_feats, Wu, col(bu), col(gu), col(betau))

    NB1 = 1024
    g1grid = (B, n // NB1)
    nbk = n // NB1
    gidx, wght = pl.pallas_call(
        functools.partial(_p1a_body, m),
        grid=g1grid,
        in_specs=[
            pl.BlockSpec((1, 3, NB1), lambda b, j: (b, 0, j)),
            pl.BlockSpec((1, m, 3), lambda b, j: (b, 0, 0)),
        ],
        out_specs=[
            pl.BlockSpec((3, NB1), lambda b, j, _n=nbk: (0, b * _n + j)),
            pl.BlockSpec((3, NB1), lambda b, j, _n=nbk: (0, b * _n + j)),
        ],
        out_shape=[
            jax.ShapeDtypeStruct((3, N), jnp.int32),
            jax.ShapeDtypeStruct((3, N), f32),
        ],
    )(un_t, known)

    tab = jnp.transpose(known_feats, (0, 2, 1)).reshape(B * m, C2)
    interp_rows = _make_sc_interp(N, C2)(tab, gidx, wght)

    y1, sc1, sh1 = pl.pallas_call(
        functools.partial(_p1b_body, g1grid[0] * g1grid[1], float(N)),
        grid=g1grid,
        in_specs=[
            pl.BlockSpec((NB1, C2), lambda b, j, _n=nbk: (b * _n + j, 0)),
            pl.BlockSpec((1, C1, NB1), lambda b, j: (b, 0, j)),
            pl.BlockSpec((C2, C1), lambda b, j: (0, 0)),
            pl.BlockSpec((C2, 1), lambda b, j: (0, 0)),
            pl.BlockSpec((C2, C2), lambda b, j: (0, 0)),
            pl.BlockSpec((C2, 1), lambda b, j: (0, 0)),
            pl.BlockSpec((C2, 1), lambda b, j: (0, 0)),
            pl.BlockSpec((C2, 1), lambda b, j: (0, 0)),
        ],
        out_specs=[
            pl.BlockSpec((1, C2, NB1), lambda b, j: (b, 0, j)),
            pl.BlockSpec((C2, 1), lambda b, j: (0, 0)),
            pl.BlockSpec((C2, 1), lambda b, j: (0, 0)),
        ],
        out_shape=[
            jax.ShapeDtypeStruct((B, C2, n), f32),
            jax.ShapeDtypeStruct((C2, 1), f32),
            jax.ShapeDtypeStruct((C2, 1), f32),
        ],
        scratch_shapes=[
            pltpu.VMEM((C2, 1), f32),
            pltpu.VMEM((C2, 1), f32),
        ],
    )(interp_rows, unknow_feats, A_u, c_u, W1, col(b1),
      col(g1), col(beta1))

    NB2 = 4096
    g2grid = (B, n // NB2)
    y2, sc2, sh2 = pl.pallas_call(
        functools.partial(_p2_body, g2grid[0] * g2grid[1], float(N)),
        grid=g2grid,
        in_specs=[
            pl.BlockSpec((1, C2, NB2), lambda b, j: (b, 0, j)),
            pl.BlockSpec((C2, 1), lambda b, j: (0, 0)),
            pl.BlockSpec((C2, 1), lambda b, j: (0, 0)),
            pl.BlockSpec((C2, C2), lambda b, j: (0, 0)),
            pl.BlockSpec((C2, 1), lambda b, j: (0, 0)),
            pl.BlockSpec((C2, 1), lambda b, j: (0, 0)),
            pl.BlockSpec((C2, 1), lambda b, j: (0, 0)),
        ],
        out_specs=[
            pl.BlockSpec((1, C2, NB2), lambda b, j: (b, 0, j)),
            pl.BlockSpec((C2, 1), lambda b, j: (0, 0)),
            pl.BlockSpec((C2, 1), lambda b, j: (0, 0)),
        ],
        out_shape=[
            jax.ShapeDtypeStruct((B, C2, n), f32),
            jax.ShapeDtypeStruct((C2, 1), f32),
            jax.ShapeDtypeStruct((C2, 1), f32),
        ],
        scratch_shapes=[
            pltpu.VMEM((C2, 1), f32),
            pltpu.VMEM((C2, 1), f32),
        ],
    )(y1, sc1, sh1, W2, col(b2), col(g2), col(beta2))

    NB3 = 4096
    out = pl.pallas_call(
        _p3_body,
        grid=(B, n // NB3),
        in_specs=[
            pl.BlockSpec((1, C2, NB3), lambda b, j: (b, 0, j)),
            pl.BlockSpec((C2, 1), lambda b, j: (0, 0)),
            pl.BlockSpec((C2, 1), lambda b, j: (0, 0)),
        ],
        out_specs=pl.BlockSpec((1, C2, NB3), lambda b, j: (b, 0, j)),
        out_shape=jax.ShapeDtypeStruct((B, C2, n), f32),
    )(y2, sc2, sh2)

    return out


# speedup vs baseline: 1.1911x; 1.1911x over previous
import functools

import jax
import jax.numpy as jnp
from jax import lax
from jax.experimental import pallas as pl
from jax.experimental.pallas import tpu as pltpu
from jax.experimental.pallas import tpu_sc as plsc

_EPS_BN = 1e-5
_EPS_D = 1e-8


def _p0_body(nsteps, N, uf_ref, Wu_ref, bu_ref, gu_ref, betau_ref,
             A_ref, c_ref, s1_ref, s2_ref):
    i = pl.program_id(0) * pl.num_programs(1) + pl.program_id(1)
    blk = uf_ref[0]

    @pl.when(i == 0)
    def _init():
        s1_ref[...] = jnp.zeros_like(s1_ref)
        s2_ref[...] = jnp.zeros_like(s2_ref)

    s1_ref[...] += jnp.sum(blk, axis=1, keepdims=True)
    s2_ref[...] += lax.dot_general(blk, blk, (((1,), (1,)), ((), ())),
                                   preferred_element_type=jnp.float32)

    @pl.when(i == nsteps - 1)
    def _fin():
        Wu = Wu_ref[...]
        mean_uf = s1_ref[...] / N
        mu_lin = jnp.dot(Wu, mean_uf, preferred_element_type=jnp.float32)
        e2 = jnp.sum(
            jnp.dot(Wu, s2_ref[...], preferred_element_type=jnp.float32) * Wu,
            axis=1, keepdims=True) / N
        var = e2 - mu_lin * mu_lin
        scale = gu_ref[...] * lax.rsqrt(var + _EPS_BN)
        A_ref[...] = Wu * scale
        c_ref[...] = betau_ref[...] - mu_lin * scale


def _p1_body(nsteps, N, m,
             un_ref, kn_ref, uf_ref, kf_ref,
             A_ref, cu_ref, W1_ref, b1_ref, g1_ref, beta1_ref,
             y1_ref, sc1_ref, sh1_ref, s1_ref, s2_ref):
    step = pl.program_id(0) * pl.num_programs(1) + pl.program_id(1)
    U = un_ref[0]
    K = kn_ref[0]
    e0 = K[:, 0:1] - U[0:1, :]
    e1 = K[:, 1:2] - U[1:2, :]
    e2 = K[:, 2:3] - U[2:3, :]
    d2 = e0 * e0 + e1 * e1 + e2 * e2

    rio = lax.broadcasted_iota(jnp.int32, d2.shape, 0)
    kk = (lax.bitcast_convert_type(d2, jnp.int32) & jnp.int32(-1024)) | rio
    vals, idxs = [], []
    for _ in range(3):
        kmin = jnp.min(kk, axis=0, keepdims=True)
        idxs.append(kmin & jnp.int32(1023))
        vals.append(lax.bitcast_convert_type(kmin & jnp.int32(-1024),
                                             jnp.float32))
        kk = jnp.where(kk == kmin, jnp.int32(2147483647), kk)

    r = [1.0 / (v + _EPS_D) for v in vals]
    norm = r[0] + r[1] + r[2]
    w = [ri / norm for ri in r]

    onehot = (jnp.where(rio == idxs[0], w[0], 0.0)
              + jnp.where(rio == idxs[1], w[1], 0.0)
              + jnp.where(rio == idxs[2], w[2], 0.0))
    interp = jnp.dot(kf_ref[0], onehot,
                     preferred_element_type=jnp.float32)

    x = interp + jnp.dot(A_ref[...], uf_ref[0],
                         preferred_element_type=jnp.float32) + cu_ref[...]
    y1 = jnp.dot(W1_ref[...], x,
                 preferred_element_type=jnp.float32) + b1_ref[...]
    y1_ref[0] = y1

    @pl.when(step == 0)
    def _init():
        s1_ref[...] = jnp.zeros_like(s1_ref)
        s2_ref[...] = jnp.zeros_like(s2_ref)

    s1_ref[...] += jnp.sum(y1, axis=1, keepdims=True)
    s2_ref[...] += jnp.sum(y1 * y1, axis=1, keepdims=True)

    @pl.when(step == nsteps - 1)
    def _fin():
        mean = s1_ref[...] / N
        var = s2_ref[...] / N - mean * mean
        scale = g1_ref[...] * lax.rsqrt(var + _EPS_BN)
        sc1_ref[...] = scale
        sh1_ref[...] = beta1_ref[...] - mean * scale


def _p1a_body(m, un_ref, kn_ref, gidx_ref, w_ref):
    b = pl.program_id(0)
    U = un_ref[0]
    K = kn_ref[0]
    e0 = K[:, 0:1] - U[0:1, :]
    e1 = K[:, 1:2] - U[1:2, :]
    e2 = K[:, 2:3] - U[2:3, :]
    d2 = e0 * e0 + e1 * e1 + e2 * e2

    rio = lax.broadcasted_iota(jnp.int32, d2.shape, 0)
    kk = (lax.bitcast_convert_type(d2, jnp.int32) & jnp.int32(-1024)) | rio
    vals, idxs = [], []
    for _ in range(3):
        kmin = jnp.min(kk, axis=0, keepdims=True)
        idxs.append(kmin & jnp.int32(1023))
        vals.append(lax.bitcast_convert_type(kmin & jnp.int32(-1024),
                                             jnp.float32))
        kk = jnp.where(kk == kmin, jnp.int32(2147483647), kk)

    r = [1.0 / (v + _EPS_D) for v in vals]
    norm = r[0] + r[1] + r[2]
    gidx_ref[...] = jnp.concatenate(idxs, axis=0) + b * m
    w_ref[...] = jnp.concatenate([ri / norm for ri in r], axis=0)


_SC_NW = 32
_SC_G = 64


def _make_sc_interp(N, C2):
    QW = N // _SC_NW
    G = _SC_G
    mesh = plsc.VectorSubcoreMesh(core_axis_name="c", subcore_axis_name="s")
    f32, i32 = jnp.float32, jnp.int32

    NCH = QW // G

    @functools.partial(
        pl.kernel, mesh=mesh,
        out_type=jax.ShapeDtypeStruct((N, C2), f32),
        scratch_types=[
            pltpu.VMEM((2, G), i32), pltpu.VMEM((2, G), i32),
            pltpu.VMEM((2, G), i32),
            pltpu.VMEM((2, G + 16), f32), pltpu.VMEM((2, G + 16), f32),
            pltpu.VMEM((2, G + 16), f32),
            pltpu.VMEM((G, C2), f32), pltpu.VMEM((G, C2), f32),
            pltpu.VMEM((G, C2), f32),
            pltpu.VMEM((G, C2), f32), pltpu.VMEM((G, C2), f32),
            pltpu.VMEM((G, C2), f32),
            pltpu.SemaphoreType.DMA, pltpu.SemaphoreType.DMA,
        ],
    )
    def sc_interp(tab_hbm, gidx_hbm, w_hbm, out_hbm,
                  i0_v, i1_v, i2_v, w0_v, w1_v, w2_v,
                  r0a, r1a, r2a, r0b, r1b, r2b, sema, semb):
        wid = lax.axis_index("s") * 2 + lax.axis_index("c")
        base = wid * QW
        bufs = ((r0a, r1a, r2a, sema), (r0b, r1b, r2b, semb))

        def issue(ci, s):
            q0 = base + ci * G
            r0, r1, r2, sem = bufs[s]
            pltpu.sync_copy(gidx_hbm.at[0, pl.ds(q0, G)], i0_v.at[s])
            pltpu.sync_copy(gidx_hbm.at[1, pl.ds(q0, G)], i1_v.at[s])
            pltpu.sync_copy(gidx_hbm.at[2, pl.ds(q0, G)], i2_v.at[s])
            pltpu.sync_copy(w_hbm.at[0, pl.ds(q0, G)],
                            w0_v.at[s, pl.ds(0, G)])
            pltpu.sync_copy(w_hbm.at[1, pl.ds(q0, G)],
                            w1_v.at[s, pl.ds(0, G)])
            pltpu.sync_copy(w_hbm.at[2, pl.ds(q0, G)],
                            w2_v.at[s, pl.ds(0, G)])
            pltpu.async_copy(tab_hbm.at[i0_v.at[s]], r0, sem)
            pltpu.async_copy(tab_hbm.at[i1_v.at[s]], r1, sem)
            pltpu.async_copy(tab_hbm.at[i2_v.at[s]], r2, sem)

        def consume(ci, s):
            q0 = base + ci * G
            r0, r1, r2, sem = bufs[s]
            pltpu.make_async_copy(tab_hbm.at[i0_v.at[s]], r0, sem).wait()
            pltpu.make_async_copy(tab_hbm.at[i1_v.at[s]], r1, sem).wait()
            pltpu.make_async_copy(tab_hbm.at[i2_v.at[s]], r2, sem).wait()

            def per_q(q, c2):
                w0 = w0_v[s, pl.ds(q, 16)][0]
                w1 = w1_v[s, pl.ds(q, 16)][0]
                w2 = w2_v[s, pl.ds(q, 16)][0]
                for c in range(C2 // 16):
                    sl = pl.ds(c * 16, 16)
                    r0[q, sl] = (w0 * r0[q, sl] + w1 * r1[q, sl]
                                 + w2 * r2[q, sl])
                return c2

            lax.fori_loop(0, G, per_q, 0)
            pltpu.sync_copy(r0, out_hbm.at[pl.ds(q0, G)])

        issue(0, 0)
        issue(1, 1)

        def pair(i, carry):
            consume(2 * i, 0)

            @pl.when(2 * i + 2 < NCH)
            def _():
                issue(2 * i + 2, 0)

            consume(2 * i + 1, 1)

            @pl.when(2 * i + 3 < NCH)
            def _():
                issue(2 * i + 3, 1)

            return carry

        lax.fori_loop(0, NCH // 2, pair, 0)

    return sc_interp


def _p1b_body(nsteps, N, it_ref, uf_ref,
              A_ref, cu_ref, W1_ref, b1_ref, g1_ref, beta1_ref,
              y1_ref, sc1_ref, sh1_ref, s1_ref, s2_ref):
    step = pl.program_id(0) * pl.num_programs(1) + pl.program_id(1)
    interp = jnp.transpose(it_ref[...])
    x = interp + jnp.dot(A_ref[...], uf_ref[0],
                         preferred_element_type=jnp.float32) + cu_ref[...]
    y1 = jnp.dot(W1_ref[...], x,
                 preferred_element_type=jnp.float32) + b1_ref[...]
    y1_ref[0] = y1

    @pl.when(step == 0)
    def _init():
        s1_ref[...] = jnp.zeros_like(s1_ref)
        s2_ref[...] = jnp.zeros_like(s2_ref)

    s1_ref[...] += jnp.sum(y1, axis=1, keepdims=True)
    s2_ref[...] += jnp.sum(y1 * y1, axis=1, keepdims=True)

    @pl.when(step == nsteps - 1)
    def _fin():
        mean = s1_ref[...] / N
        var = s2_ref[...] / N - mean * mean
        scale = g1_ref[...] * lax.rsqrt(var + _EPS_BN)
        sc1_ref[...] = scale
        sh1_ref[...] = beta1_ref[...] - mean * scale


def _p2_body(nsteps, N, y1_ref, sc1_ref, sh1_ref, W2_ref, b2_ref,
             g2_ref, beta2_ref, y2_ref, sc2_ref, sh2_ref, s1_ref, s2_ref):
    step = pl.program_id(0) * pl.num_programs(1) + pl.program_id(1)
    x1 = jnp.maximum(y1_ref[0] * sc1_ref[...] + sh1_ref[...], 0.0)
    y2 = jnp.dot(W2_ref[...], x1,
                 preferred_element_type=jnp.float32) + b2_ref[...]
    y2_ref[0] = y2

    @pl.when(step == 0)
    def _init():
        s1_ref[...] = jnp.zeros_like(s1_ref)
        s2_ref[...] = jnp.zeros_like(s2_ref)

    s1_ref[...] += jnp.sum(y2, axis=1, keepdims=True)
    s2_ref[...] += jnp.sum(y2 * y2, axis=1, keepdims=True)

    @pl.when(step == nsteps - 1)
    def _fin():
        mean = s1_ref[...] / N
        var = s2_ref[...] / N - mean * mean
        scale = g2_ref[...] * lax.rsqrt(var + _EPS_BN)
        sc2_ref[...] = scale
        sh2_ref[...] = beta2_ref[...] - mean * scale


def _p3_body(y2_ref, sc2_ref, sh2_ref, out_ref):
    out_ref[0] = jnp.maximum(y2_ref[0] * sc2_ref[...] + sh2_ref[...], 0.0)


def kernel(unknown, known, unknow_feats, known_feats, Wu, bu, gu, betau,
           W1, b1, g1, beta1, W2, b2, g2, beta2):
    B, n, _ = unknown.shape
    m = known.shape[1]
    C1 = unknow_feats.shape[1]
    C2 = known_feats.shape[1]
    N = B * n
    f32 = jnp.float32

    un_t = jnp.transpose(unknown, (0, 2, 1))
    col = lambda v: v.reshape(-1, 1)

    NB0 = 4096
    g0 = (B, n // NB0)
    A_u, c_u = pl.pallas_call(
        functools.partial(_p0_body, g0[0] * g0[1], float(N)),
        grid=g0,
        in_specs=[
            pl.BlockSpec((1, C1, NB0), lambda b, j: (b, 0, j)),
            pl.BlockSpec((C2, C1), lambda b, j: (0, 0)),
            pl.BlockSpec((C2, 1), lambda b, j: (0, 0)),
            pl.BlockSpec((C2, 1), lambda b, j: (0, 0)),
            pl.BlockSpec((C2, 1), lambda b, j: (0, 0)),
        ],
        out_specs=[
            pl.BlockSpec((C2, C1), lambda b, j: (0, 0)),
            pl.BlockSpec((C2, 1), lambda b, j: (0, 0)),
        ],
        out_shape=[
            jax.ShapeDtypeStruct((C2, C1), f32),
            jax.ShapeDtypeStruct((C2, 1), f32),
        ],
        scratch_shapes=[
            pltpu.VMEM((C1, 1), f32),
            pltpu.VMEM((C1, C1), f32),
        ],
    )(unknow_feats, Wu, col(bu), col(gu), col(betau))

    NB1 = 1024
    g1grid = (B, n // NB1)
    nbk = n // NB1
    gidx, wght = pl.pallas_call(
        functools.partial(_p1a_body, m),
        grid=g1grid,
        in_specs=[
            pl.BlockSpec((1, 3, NB1), lambda b, j: (b, 0, j)),
            pl.BlockSpec((1, m, 3), lambda b, j: (b, 0, 0)),
        ],
        out_specs=[
            pl.BlockSpec((3, NB1), lambda b, j, _n=nbk: (0, b * _n + j)),
            pl.BlockSpec((3, NB1), lambda b, j, _n=nbk: (0, b * _n + j)),
        ],
        out_shape=[
            jax.ShapeDtypeStruct((3, N), jnp.int32),
            jax.ShapeDtypeStruct((3, N), f32),
        ],
    )(un_t, known)

    tab = jnp.transpose(known_feats, (0, 2, 1)).reshape(B * m, C2)
    interp_rows = _make_sc_interp(N, C2)(tab, gidx, wght)

    y1, sc1, sh1 = pl.pallas_call(
        functools.partial(_p1b_body, g1grid[0] * g1grid[1], float(N)),
        grid=g1grid,
        in_specs=[
            pl.BlockSpec((NB1, C2), lambda b, j, _n=nbk: (b * _n + j, 0)),
            pl.BlockSpec((1, C1, NB1), lambda b, j: (b, 0, j)),
            pl.BlockSpec((C2, C1), lambda b, j: (0, 0)),
            pl.BlockSpec((C2, 1), lambda b, j: (0, 0)),
            pl.BlockSpec((C2, C2), lambda b, j: (0, 0)),
            pl.BlockSpec((C2, 1), lambda b, j: (0, 0)),
            pl.BlockSpec((C2, 1), lambda b, j: (0, 0)),
            pl.BlockSpec((C2, 1), lambda b, j: (0, 0)),
        ],
        out_specs=[
            pl.BlockSpec((1, C2, NB1), lambda b, j: (b, 0, j)),
            pl.BlockSpec((C2, 1), lambda b, j: (0, 0)),
            pl.BlockSpec((C2, 1), lambda b, j: (0, 0)),
        ],
        out_shape=[
            jax.ShapeDtypeStruct((B, C2, n), f32),
            jax.ShapeDtypeStruct((C2, 1), f32),
            jax.ShapeDtypeStruct((C2, 1), f32),
        ],
        scratch_shapes=[
            pltpu.VMEM((C2, 1), f32),
            pltpu.VMEM((C2, 1), f32),
        ],
    )(interp_rows, unknow_feats, A_u, c_u, W1, col(b1),
      col(g1), col(beta1))

    NB2 = 4096
    g2grid = (B, n // NB2)
    y2, sc2, sh2 = pl.pallas_call(
        functools.partial(_p2_body, g2grid[0] * g2grid[1], float(N)),
        grid=g2grid,
        in_specs=[
            pl.BlockSpec((1, C2, NB2), lambda b, j: (b, 0, j)),
            pl.BlockSpec((C2, 1), lambda b, j: (0, 0)),
            pl.BlockSpec((C2, 1), lambda b, j: (0, 0)),
            pl.BlockSpec((C2, C2), lambda b, j: (0, 0)),
            pl.BlockSpec((C2, 1), lambda b, j: (0, 0)),
            pl.BlockSpec((C2, 1), lambda b, j: (0, 0)),
            pl.BlockSpec((C2, 1), lambda b, j: (0, 0)),
        ],
        out_specs=[
            pl.BlockSpec((1, C2, NB2), lambda b, j: (b, 0, j)),
            pl.BlockSpec((C2, 1), lambda b, j: (0, 0)),
            pl.BlockSpec((C2, 1), lambda b, j: (0, 0)),
        ],
        out_shape=[
            jax.ShapeDtypeStruct((B, C2, n), f32),
            jax.ShapeDtypeStruct((C2, 1), f32),
            jax.ShapeDtypeStruct((C2, 1), f32),
        ],
        scratch_shapes=[
            pltpu.VMEM((C2, 1), f32),
            pltpu.VMEM((C2, 1), f32),
        ],
    )(y1, sc1, sh1, W2, col(b2), col(g2), col(beta2))

    NB3 = 4096
    out = pl.pallas_call(
        _p3_body,
        grid=(B, n // NB3),
        in_specs=[
            pl.BlockSpec((1, C2, NB3), lambda b, j: (b, 0, j)),
            pl.BlockSpec((C2, 1), lambda b, j: (0, 0)),
            pl.BlockSpec((C2, 1), lambda b, j: (0, 0)),
        ],
        out_specs=pl.BlockSpec((1, C2, NB3), lambda b, j: (b, 0, j)),
        out_shape=jax.ShapeDtypeStruct((B, C2, n), f32),
    )(y2, sc2, sh2)

    return out


# speedup vs baseline: 1.5048x; 1.2634x over previous
import functools

import jax
import jax.numpy as jnp
from jax import lax
from jax.experimental import pallas as pl
from jax.experimental.pallas import tpu as pltpu

_EPS_BN = 1e-5
_EPS_D = 1e-8


def _p0_body(nsteps, N, uf_ref, Wu_ref, bu_ref, gu_ref, betau_ref,
             A_ref, c_ref, s1_ref, s2_ref):
    i = pl.program_id(0) * pl.num_programs(1) + pl.program_id(1)
    blk = uf_ref[0]

    @pl.when(i == 0)
    def _init():
        s1_ref[...] = jnp.zeros_like(s1_ref)
        s2_ref[...] = jnp.zeros_like(s2_ref)

    s1_ref[...] += jnp.sum(blk, axis=1, keepdims=True)
    s2_ref[...] += lax.dot_general(blk, blk, (((1,), (1,)), ((), ())),
                                   preferred_element_type=jnp.float32)

    @pl.when(i == nsteps - 1)
    def _fin():
        Wu = Wu_ref[...]
        mean_uf = s1_ref[...] / N
        mu_lin = jnp.dot(Wu, mean_uf, preferred_element_type=jnp.float32)
        e2 = jnp.sum(
            jnp.dot(Wu, s2_ref[...], preferred_element_type=jnp.float32) * Wu,
            axis=1, keepdims=True) / N
        var = e2 - mu_lin * mu_lin
        scale = gu_ref[...] * lax.rsqrt(var + _EPS_BN)
        A_ref[...] = Wu * scale
        c_ref[...] = betau_ref[...] - mu_lin * scale


def _p1_body(nsteps, N, m,
             un_ref, kn_ref, uf_ref, kf_ref,
             A_ref, cu_ref, W1_ref, b1_ref, g1_ref, beta1_ref,
             y1_ref, sc1_ref, sh1_ref, s1_ref, s2_ref):
    step = pl.program_id(0) * pl.num_programs(1) + pl.program_id(1)
    U = un_ref[0]
    K = kn_ref[0]
    e0 = K[:, 0:1] - U[0:1, :]
    e1 = K[:, 1:2] - U[1:2, :]
    e2 = K[:, 2:3] - U[2:3, :]
    d2 = e0 * e0 + e1 * e1 + e2 * e2

    rio = lax.broadcasted_iota(jnp.int32, d2.shape, 0)
    kk = (lax.bitcast_convert_type(d2, jnp.int32) & jnp.int32(-1024)) | rio
    vals, idxs = [], []
    for _ in range(3):
        kmin = jnp.min(kk, axis=0, keepdims=True)
        idxs.append(kmin & jnp.int32(1023))
        vals.append(lax.bitcast_convert_type(kmin & jnp.int32(-1024),
                                             jnp.float32))
        kk = jnp.where(kk == kmin, jnp.int32(2147483647), kk)

    r = [1.0 / (v + _EPS_D) for v in vals]
    norm = r[0] + r[1] + r[2]
    bf16 = jnp.bfloat16
    w = [ri / norm for ri in r]

    onehot = (jnp.where(rio == idxs[0], w[0], 0.0)
              + jnp.where(rio == idxs[1], w[1], 0.0)
              + jnp.where(rio == idxs[2], w[2], 0.0))
    interp = jnp.dot(kf_ref[0], onehot.astype(bf16),
                     preferred_element_type=jnp.float32)

    x = interp + jnp.dot(A_ref[...].astype(bf16), uf_ref[0],
                         preferred_element_type=jnp.float32) + cu_ref[...]
    y1 = jnp.dot(W1_ref[...].astype(bf16), x.astype(bf16),
                 preferred_element_type=jnp.float32) + b1_ref[...]
    y1_ref[0] = y1.astype(bf16)

    @pl.when(step == 0)
    def _init():
        s1_ref[...] = jnp.zeros_like(s1_ref)
        s2_ref[...] = jnp.zeros_like(s2_ref)

    s1_ref[...] += jnp.sum(y1, axis=1, keepdims=True)
    s2_ref[...] += jnp.sum(y1 * y1, axis=1, keepdims=True)

    @pl.when(step == nsteps - 1)
    def _fin():
        mean = s1_ref[...] / N
        var = s2_ref[...] / N - mean * mean
        scale = g1_ref[...] * lax.rsqrt(var + _EPS_BN)
        sc1_ref[...] = scale
        sh1_ref[...] = beta1_ref[...] - mean * scale


def _p2_body(nsteps, N, y1_ref, sc1_ref, sh1_ref, W2_ref, b2_ref,
             g2_ref, beta2_ref, y2_ref, sc2_ref, sh2_ref, s1_ref, s2_ref):
    step = pl.program_id(0) * pl.num_programs(1) + pl.program_id(1)
    x1 = jnp.maximum(
        y1_ref[0].astype(jnp.float32) * sc1_ref[...] + sh1_ref[...], 0.0)
    y2 = jnp.dot(W2_ref[...].astype(jnp.bfloat16), x1.astype(jnp.bfloat16),
                 preferred_element_type=jnp.float32) + b2_ref[...]
    y2_ref[0] = y2.astype(jnp.bfloat16)

    @pl.when(step == 0)
    def _init():
        s1_ref[...] = jnp.zeros_like(s1_ref)
        s2_ref[...] = jnp.zeros_like(s2_ref)

    s1_ref[...] += jnp.sum(y2, axis=1, keepdims=True)
    s2_ref[...] += jnp.sum(y2 * y2, axis=1, keepdims=True)

    @pl.when(step == nsteps - 1)
    def _fin():
        mean = s1_ref[...] / N
        var = s2_ref[...] / N - mean * mean
        scale = g2_ref[...] * lax.rsqrt(var + _EPS_BN)
        sc2_ref[...] = scale
        sh2_ref[...] = beta2_ref[...] - mean * scale


def _p3_body(y2_ref, sc2_ref, sh2_ref, out_ref):
    out_ref[0] = jnp.maximum(
        y2_ref[0].astype(jnp.float32) * sc2_ref[...] + sh2_ref[...], 0.0)


def kernel(unknown, known, unknow_feats, known_feats, Wu, bu, gu, betau,
           W1, b1, g1, beta1, W2, b2, g2, beta2):
    B, n, _ = unknown.shape
    m = known.shape[1]
    C1 = unknow_feats.shape[1]
    C2 = known_feats.shape[1]
    N = B * n
    f32 = jnp.float32

    un_t = jnp.transpose(unknown, (0, 2, 1))
    col = lambda v: v.reshape(-1, 1)

    NB0 = 4096
    g0 = (B, n // NB0)
    A_u, c_u = pl.pallas_call(
        functools.partial(_p0_body, g0[0] * g0[1], float(N)),
        grid=g0,
        in_specs=[
            pl.BlockSpec((1, C1, NB0), lambda b, j: (b, 0, j)),
            pl.BlockSpec((C2, C1), lambda b, j: (0, 0)),
            pl.BlockSpec((C2, 1), lambda b, j: (0, 0)),
            pl.BlockSpec((C2, 1), lambda b, j: (0, 0)),
            pl.BlockSpec((C2, 1), lambda b, j: (0, 0)),
        ],
        out_specs=[
            pl.BlockSpec((C2, C1), lambda b, j: (0, 0)),
            pl.BlockSpec((C2, 1), lambda b, j: (0, 0)),
        ],
        out_shape=[
            jax.ShapeDtypeStruct((C2, C1), f32),
            jax.ShapeDtypeStruct((C2, 1), f32),
        ],
        scratch_shapes=[
            pltpu.VMEM((C1, 1), f32),
            pltpu.VMEM((C1, C1), f32),
        ],
    )(unknow_feats, Wu, col(bu), col(gu), col(betau))

    NB1 = 1024
    g1grid = (B, n // NB1)
    y1, sc1, sh1 = pl.pallas_call(
        functools.partial(_p1_body, g1grid[0] * g1grid[1], float(N), m),
        grid=g1grid,
        in_specs=[
            pl.BlockSpec((1, 3, NB1), lambda b, j: (b, 0, j)),
            pl.BlockSpec((1, m, 3), lambda b, j: (b, 0, 0)),
            pl.BlockSpec((1, C1, NB1), lambda b, j: (b, 0, j)),
            pl.BlockSpec((1, C2, m), lambda b, j: (b, 0, 0)),
            pl.BlockSpec((C2, C1), lambda b, j: (0, 0)),
            pl.BlockSpec((C2, 1), lambda b, j: (0, 0)),
            pl.BlockSpec((C2, C2), lambda b, j: (0, 0)),
            pl.BlockSpec((C2, 1), lambda b, j: (0, 0)),
            pl.BlockSpec((C2, 1), lambda b, j: (0, 0)),
            pl.BlockSpec((C2, 1), lambda b, j: (0, 0)),
        ],
        out_specs=[
            pl.BlockSpec((1, C2, NB1), lambda b, j: (b, 0, j)),
            pl.BlockSpec((C2, 1), lambda b, j: (0, 0)),
            pl.BlockSpec((C2, 1), lambda b, j: (0, 0)),
        ],
        out_shape=[
            jax.ShapeDtypeStruct((B, C2, n), jnp.bfloat16),
            jax.ShapeDtypeStruct((C2, 1), f32),
            jax.ShapeDtypeStruct((C2, 1), f32),
        ],
        scratch_shapes=[
            pltpu.VMEM((C2, 1), f32),
            pltpu.VMEM((C2, 1), f32),
        ],
    )(un_t, known, unknow_feats.astype(jnp.bfloat16),
      known_feats.astype(jnp.bfloat16), A_u, c_u, W1, col(b1),
      col(g1), col(beta1))

    NB2 = 4096
    g2grid = (B, n // NB2)
    y2, sc2, sh2 = pl.pallas_call(
        functools.partial(_p2_body, g2grid[0] * g2grid[1], float(N)),
        grid=g2grid,
        in_specs=[
            pl.BlockSpec((1, C2, NB2), lambda b, j: (b, 0, j)),
            pl.BlockSpec((C2, 1), lambda b, j: (0, 0)),
            pl.BlockSpec((C2, 1), lambda b, j: (0, 0)),
            pl.BlockSpec((C2, C2), lambda b, j: (0, 0)),
            pl.BlockSpec((C2, 1), lambda b, j: (0, 0)),
            pl.BlockSpec((C2, 1), lambda b, j: (0, 0)),
            pl.BlockSpec((C2, 1), lambda b, j: (0, 0)),
        ],
        out_specs=[
            pl.BlockSpec((1, C2, NB2), lambda b, j: (b, 0, j)),
            pl.BlockSpec((C2, 1), lambda b, j: (0, 0)),
            pl.BlockSpec((C2, 1), lambda b, j: (0, 0)),
        ],
        out_shape=[
            jax.ShapeDtypeStruct((B, C2, n), jnp.bfloat16),
            jax.ShapeDtypeStruct((C2, 1), f32),
            jax.ShapeDtypeStruct((C2, 1), f32),
        ],
        scratch_shapes=[
            pltpu.VMEM((C2, 1), f32),
            pltpu.VMEM((C2, 1), f32),
        ],
    )(y1, sc1, sh1, W2, col(b2), col(g2), col(beta2))

    NB3 = 4096
    out = pl.pallas_call(
        _p3_body,
        grid=(B, n // NB3),
        in_specs=[
            pl.BlockSpec((1, C2, NB3), lambda b, j: (b, 0, j)),
            pl.BlockSpec((C2, 1), lambda b, j: (0, 0)),
            pl.BlockSpec((C2, 1), lambda b, j: (0, 0)),
        ],
        out_specs=pl.BlockSpec((1, C2, NB3), lambda b, j: (b, 0, j)),
        out_shape=jax.ShapeDtypeStruct((B, C2, n), f32),
    )(y2, sc2, sh2)

    return out
